# baseline (speedup 1.0000x reference)
import functools

import jax
import jax.numpy as jnp
from jax import lax
from jax.experimental import pallas as pl
from jax.experimental.pallas import tpu as pltpu
from jax.experimental.pallas import tpu_sc as plsc

_NUM_FIELDS = 26
_VOCAB = 100000
_EMBED_DIM = 16
_NUM_DENSE = 13
_BATCH = 16384

_NC = 2
_NS = 16
_NW = _NC * _NS

_CHUNK_COLS = 1408
_CHUNKS_PER_FIELD = 71
_TOTAL_CHUNKS = _NUM_FIELDS * _CHUNKS_PER_FIELD
_SLOTS = (_TOTAL_CHUNKS + _NW - 1) // _NW
_CHUNK_ELEMS = _CHUNK_COLS * _EMBED_DIM
_CHUNK_J = _CHUNK_ELEMS // 128
_ELEMS_PER_FIELD = _VOCAB * _EMBED_DIM
_VOCAB_MAIN = _CHUNKS_PER_FIELD * _CHUNK_COLS
_SIDE_COLS = _VOCAB - _VOCAB_MAIN
_SIDE_ELEMS_PER_FIELD = _SIDE_COLS * _EMBED_DIM
_SIDE_ELEMS = _NUM_FIELDS * _SIDE_ELEMS_PER_FIELD
_RM_ELEMS = _NUM_FIELDS * _VOCAB * _EMBED_DIM


def _transpose_chunk(slab, buf):
    ibase = lax.iota(jnp.int32, 16) * _EMBED_DIM

    @plsc.parallel_loop(0, _CHUNK_COLS // 16, unroll=2)
    def _(c0):
        for i in range(16):
            vec = slab[i, pl.ds(16 * c0, 16)]
            idx = ibase + (256 * c0 + i)
            plsc.store_scatter(buf, [idx], vec)


def _reformat_body(tbl_hbm, tail_hbm, rm_hbm, slab0, slab1, buf0, buf1,
                   side_v, isem0, isem1, osem0, osem1):
    wid = lax.axis_index("s") * _NC + lax.axis_index("c")

    @pl.when(wid == _NW - 1)
    def _():
        pltpu.sync_copy(tail_hbm, side_v)

        def side(f, carry):
            pltpu.sync_copy(
                side_v.at[pl.ds(f * _SIDE_ELEMS_PER_FIELD,
                                _SIDE_ELEMS_PER_FIELD)],
                rm_hbm.at[pl.ds(f * _ELEMS_PER_FIELD
                                + _VOCAB_MAIN * _EMBED_DIM,
                                _SIDE_ELEMS_PER_FIELD)])
            return carry

        lax.fori_loop(0, _NUM_FIELDS, side, 0)

    def chunk_id(t):
        return lax.rem(wid + _NW * t, _TOTAL_CHUNKS)

    def tbl_src(c):
        f = c // _CHUNKS_PER_FIELD
        k = lax.rem(c, _CHUNKS_PER_FIELD)
        return tbl_hbm.at[pl.ds(16 * f, 16),
                          pl.ds(_CHUNK_COLS * k, _CHUNK_COLS)]

    def rm_dst(c):
        f = c // _CHUNKS_PER_FIELD
        k = lax.rem(c, _CHUNKS_PER_FIELD)
        return rm_hbm.at[pl.ds(f * _ELEMS_PER_FIELD + _CHUNK_ELEMS * k,
                               _CHUNK_ELEMS)]

    def wait_in(slab, sem):
        pltpu.make_async_copy(
            tbl_hbm.at[pl.ds(0, 16), pl.ds(0, _CHUNK_COLS)], slab, sem).wait()

    def wait_out(buf, sem):
        pltpu.make_async_copy(
            buf, rm_hbm.at[pl.ds(0, _CHUNK_ELEMS)], sem).wait()

    pltpu.async_copy(tbl_src(chunk_id(0)), slab0, isem0)
    pltpu.async_copy(buf0, rm_dst(chunk_id(0)), osem0)
    pltpu.async_copy(buf1, rm_dst(chunk_id(1)), osem1)

    def pair(g, carry):
        t0 = 2 * g
        pltpu.async_copy(tbl_src(chunk_id(t0 + 1)), slab1, isem1)
        wait_in(slab0, isem0)
        wait_out(buf0, osem0)
        _transpose_chunk(slab0, buf0)
        pltpu.async_copy(buf0, rm_dst(chunk_id(t0)), osem0)

        @pl.when(t0 + 2 < _SLOTS)
        def _():
            pltpu.async_copy(tbl_src(chunk_id(t0 + 2)), slab0, isem0)

        wait_in(slab1, isem1)
        wait_out(buf1, osem1)
        _transpose_chunk(slab1, buf1)
        pltpu.async_copy(buf1, rm_dst(chunk_id(t0 + 1)), osem1)
        return carry

    lax.fori_loop(0, _SLOTS // 2, pair, 0)
    wait_out(buf0, osem0)
    wait_out(buf1, osem1)


@functools.cache
def _reformat_call():
    return pl.kernel(
        _reformat_body,
        mesh=plsc.VectorSubcoreMesh(core_axis_name="c", subcore_axis_name="s"),
        out_type=jax.ShapeDtypeStruct((_RM_ELEMS,), jnp.float32),
        scratch_types=[
            pltpu.VMEM((16, _CHUNK_COLS), jnp.float32),
            pltpu.VMEM((16, _CHUNK_COLS), jnp.float32),
            pltpu.VMEM((_CHUNK_ELEMS,), jnp.float32),
            pltpu.VMEM((_CHUNK_ELEMS,), jnp.float32),
            pltpu.VMEM((_SIDE_ELEMS,), jnp.float32),
            pltpu.SemaphoreType.DMA,
            pltpu.SemaphoreType.DMA,
            pltpu.SemaphoreType.DMA,
            pltpu.SemaphoreType.DMA,
        ],
        compiler_params=pltpu.CompilerParams(needs_layout_passes=False),
    )


_TOTAL_ROWS = _BATCH * _NUM_FIELDS
_ROWS_PER_W = _TOTAL_ROWS // _NW
_IDX_CHUNK = 128
_CHUNKS_PER_W = _ROWS_PER_W // _IDX_CHUNK
_FIRE = 13
_GROUPS = _CHUNKS_PER_W // _FIRE
_GROUP_ROWS = _FIRE * _IDX_CHUNK


def _gather_body(idx_hbm, table_hbm, out_hbm, idx_v, rows0, rows1,
                 sem, osem0, osem1):
    wid = lax.axis_index("s") * _NC + lax.axis_index("c")
    pltpu.sync_copy(idx_hbm.at[wid], idx_v)

    def out_at(g):
        return out_hbm.at[pl.ds(wid * _ROWS_PER_W + g * _GROUP_ROWS,
                                _GROUP_ROWS)]

    def wait_out(rows, osem):
        pltpu.make_async_copy(
            rows, out_hbm.at[pl.ds(0, _GROUP_ROWS)], osem).wait()

    def fire_and_drain(g, rows):
        copies = []
        for j in range(_FIRE):
            copies.append(
                pltpu.async_copy(
                    table_hbm.at[idx_v.at[g * _FIRE + j]],
                    rows.at[pl.ds(j * _IDX_CHUNK, _IDX_CHUNK)],
                    sem,
                )
            )
        for c in copies:
            c.wait()

    pltpu.async_copy(rows0, out_at(0), osem0)
    pltpu.async_copy(rows1, out_at(1), osem1)

    def pair(p, carry):
        g0 = 2 * p
        wait_out(rows0, osem0)
        fire_and_drain(g0, rows0)
        pltpu.async_copy(rows0, out_at(g0), osem0)
        wait_out(rows1, osem1)
        fire_and_drain(g0 + 1, rows1)
        pltpu.async_copy(rows1, out_at(g0 + 1), osem1)
        return carry

    lax.fori_loop(0, _GROUPS // 2, pair, 0)
    g_last = _GROUPS - 1
    wait_out(rows0, osem0)
    fire_and_drain(g_last, rows0)
    pltpu.async_copy(rows0, out_at(g_last), osem0)
    wait_out(rows0, osem0)
    wait_out(rows1, osem1)


@functools.cache
def _gather_call():
    return pl.kernel(
        _gather_body,
        mesh=plsc.VectorSubcoreMesh(core_axis_name="c", subcore_axis_name="s"),
        out_type=jax.ShapeDtypeStruct((_TOTAL_ROWS, _EMBED_DIM), jnp.float32),
        scratch_types=[
            pltpu.VMEM((_CHUNKS_PER_W, _IDX_CHUNK), jnp.int32),
            pltpu.VMEM((_GROUP_ROWS, _EMBED_DIM), jnp.float32),
            pltpu.VMEM((_GROUP_ROWS, _EMBED_DIM), jnp.float32),
            pltpu.SemaphoreType.DMA,
            pltpu.SemaphoreType.DMA,
            pltpu.SemaphoreType.DMA,
        ],
        compiler_params=pltpu.CompilerParams(use_tc_tiling_on_sc=False),
    )


def _mlp_body(dense_ref, emb_ref, w1a_ref, w1b_ref, b1_ref, w2_ref, b2_ref,
              w3_ref, b3_ref, wf_ref, bfw_ref, ww_ref, out_ref):
    f32 = jnp.float32
    hp = jax.lax.Precision.DEFAULT
    dense = dense_ref[...]
    h = jnp.dot(dense, w1a_ref[...], precision=hp, preferred_element_type=f32)
    h = h + jnp.dot(emb_ref[...], w1b_ref[...], precision=hp,
                    preferred_element_type=f32)
    h = jnp.maximum(h + b1_ref[...], 0.0)
    h = jnp.maximum(
        jnp.dot(h, w2_ref[...], precision=hp, preferred_element_type=f32)
        + b2_ref[...], 0.0)
    h = jnp.maximum(
        jnp.dot(h, w3_ref[...], precision=hp, preferred_element_type=f32)
        + b3_ref[...], 0.0)
    deep = jnp.sum(h * wf_ref[...], axis=1, keepdims=True)
    wide = jnp.sum(dense * ww_ref[...], axis=1, keepdims=True)
    z = 0.5 * (deep + wide + bfw_ref[...])
    out_ref[...] = jax.nn.sigmoid(z)


def _mlp_call(dense, emb_flat, w1a, w1b, b1, w2, b2, w3, b3, wfT, bfw, wwT):
    nblk = 8
    blk = _BATCH // nblk
    in_dim = _NUM_FIELDS * _EMBED_DIM
    full = lambda shape: pl.BlockSpec(shape, lambda i: (0, 0))
    return pl.pallas_call(
        _mlp_body,
        grid=(nblk,),
        in_specs=[
            pl.BlockSpec((blk, _NUM_DENSE), lambda i: (i, 0)),
            pl.BlockSpec((blk, in_dim), lambda i: (i, 0)),
            full((_NUM_DENSE, 256)),
            full((in_dim, 256)),
            full((1, 256)),
            full((256, 128)),
            full((1, 128)),
            full((128, 64)),
            full((1, 64)),
            full((1, 64)),
            full((1, 1)),
            full((1, _NUM_DENSE)),
        ],
        out_specs=pl.BlockSpec((blk, 1), lambda i: (i, 0)),
        out_shape=jax.ShapeDtypeStruct((_BATCH, 1), jnp.float32),
    )(dense, emb_flat, w1a, w1b, b1, w2, b2, w3, b3, wfT, bfw, wwT)


def kernel(dense_inputs, sparse_inputs, embed_tables,
           W1, b1, W2, b2, W3, b3, Wf, bf, Ww, bw):
    tblT = embed_tables.transpose(0, 2, 1).reshape(
        _NUM_FIELDS * _EMBED_DIM, _VOCAB)
    tail = embed_tables[:, _VOCAB_MAIN:, :].reshape(_SIDE_ELEMS)
    rm = _reformat_call()(tblT, tail)
    table = rm.reshape(_NUM_FIELDS * _VOCAB, _EMBED_DIM)

    offs = (jnp.arange(_NUM_FIELDS, dtype=jnp.int32) * _VOCAB)[None, :]
    idx = (sparse_inputs.astype(jnp.int32) + offs).reshape(
        _NW, _CHUNKS_PER_W, _IDX_CHUNK)

    emb_rows = _gather_call()(idx, table)
    emb_flat = emb_rows.reshape(_BATCH, _NUM_FIELDS * _EMBED_DIM)

    w1a = W1[:_NUM_DENSE]
    w1b = W1[_NUM_DENSE:]
    bfw = (bf + bw).reshape(1, 1)
    out = _mlp_call(dense_inputs, emb_flat,
                    w1a, w1b, b1.reshape(1, -1),
                    W2, b2.reshape(1, -1),
                    W3, b3.reshape(1, -1),
                    Wf.reshape(1, -1), bfw, Ww.reshape(1, -1))
    return out


# speedup vs baseline: 1.0076x; 1.0076x over previous
import functools

import jax
import jax.numpy as jnp
from jax import lax
from jax.experimental import pallas as pl
from jax.experimental.pallas import tpu as pltpu
from jax.experimental.pallas import tpu_sc as plsc

_NUM_FIELDS = 26
_VOCAB = 100000
_EMBED_DIM = 16
_NUM_DENSE = 13
_BATCH = 16384

_NC = 2
_NS = 16
_NW = _NC * _NS

_CHUNK_COLS = 1408
_CHUNKS_PER_FIELD = 71
_TOTAL_CHUNKS = _NUM_FIELDS * _CHUNKS_PER_FIELD
_SLOTS = (_TOTAL_CHUNKS + _NW - 1) // _NW
_CHUNK_ELEMS = _CHUNK_COLS * _EMBED_DIM
_CHUNK_J = _CHUNK_ELEMS // 128
_ELEMS_PER_FIELD = _VOCAB * _EMBED_DIM
_VOCAB_MAIN = _CHUNKS_PER_FIELD * _CHUNK_COLS
_SIDE_COLS = _VOCAB - _VOCAB_MAIN
_SIDE_ELEMS_PER_FIELD = _SIDE_COLS * _EMBED_DIM
_SIDE_ELEMS = _NUM_FIELDS * _SIDE_ELEMS_PER_FIELD
_RM_ELEMS = _NUM_FIELDS * _VOCAB * _EMBED_DIM


def _transpose_chunk(slab, buf):
    ibase = lax.iota(jnp.int32, 16) * _EMBED_DIM

    @plsc.parallel_loop(0, _CHUNK_COLS // 16, unroll=2)
    def _(c0):
        for i in range(16):
            vec = slab[i, pl.ds(16 * c0, 16)]
            idx = ibase + (256 * c0 + i)
            plsc.store_scatter(buf, [idx], vec)


def _reformat_body(tbl_hbm, tail_hbm, rm_hbm, slab0, slab1, buf0, buf1,
                   side_v, isem0, isem1, osem0, osem1):
    wid = lax.axis_index("s") * _NC + lax.axis_index("c")

    @pl.when(wid == _NW - 1)
    def _():
        pltpu.sync_copy(tail_hbm, side_v)

        def side(f, carry):
            pltpu.sync_copy(
                side_v.at[pl.ds(f * _SIDE_ELEMS_PER_FIELD,
                                _SIDE_ELEMS_PER_FIELD)],
                rm_hbm.at[pl.ds(f * _ELEMS_PER_FIELD
                                + _VOCAB_MAIN * _EMBED_DIM,
                                _SIDE_ELEMS_PER_FIELD)])
            return carry

        lax.fori_loop(0, _NUM_FIELDS, side, 0)

    def chunk_id(t):
        return lax.rem(wid + _NW * t, _TOTAL_CHUNKS)

    def tbl_src(c):
        f = c // _CHUNKS_PER_FIELD
        k = lax.rem(c, _CHUNKS_PER_FIELD)
        return tbl_hbm.at[pl.ds(16 * f, 16),
                          pl.ds(_CHUNK_COLS * k, _CHUNK_COLS)]

    def rm_dst(c):
        f = c // _CHUNKS_PER_FIELD
        k = lax.rem(c, _CHUNKS_PER_FIELD)
        return rm_hbm.at[pl.ds(f * _ELEMS_PER_FIELD + _CHUNK_ELEMS * k,
                               _CHUNK_ELEMS)]

    def wait_in(slab, sem):
        pltpu.make_async_copy(
            tbl_hbm.at[pl.ds(0, 16), pl.ds(0, _CHUNK_COLS)], slab, sem).wait()

    def wait_out(buf, sem):
        pltpu.make_async_copy(
            buf, rm_hbm.at[pl.ds(0, _CHUNK_ELEMS)], sem).wait()

    pltpu.async_copy(tbl_src(chunk_id(0)), slab0, isem0)
    pltpu.async_copy(buf0, rm_dst(chunk_id(0)), osem0)
    pltpu.async_copy(buf1, rm_dst(chunk_id(1)), osem1)

    def pair(g, carry):
        t0 = 2 * g
        pltpu.async_copy(tbl_src(chunk_id(t0 + 1)), slab1, isem1)
        wait_in(slab0, isem0)
        wait_out(buf0, osem0)
        _transpose_chunk(slab0, buf0)
        pltpu.async_copy(buf0, rm_dst(chunk_id(t0)), osem0)

        @pl.when(t0 + 2 < _SLOTS)
        def _():
            pltpu.async_copy(tbl_src(chunk_id(t0 + 2)), slab0, isem0)

        wait_in(slab1, isem1)
        wait_out(buf1, osem1)
        _transpose_chunk(slab1, buf1)
        pltpu.async_copy(buf1, rm_dst(chunk_id(t0 + 1)), osem1)
        return carry

    lax.fori_loop(0, _SLOTS // 2, pair, 0)
    wait_out(buf0, osem0)
    wait_out(buf1, osem1)


@functools.cache
def _reformat_call():
    return pl.kernel(
        _reformat_body,
        mesh=plsc.VectorSubcoreMesh(core_axis_name="c", subcore_axis_name="s"),
        out_type=jax.ShapeDtypeStruct((_RM_ELEMS,), jnp.float32),
        scratch_types=[
            pltpu.VMEM((16, _CHUNK_COLS), jnp.float32),
            pltpu.VMEM((16, _CHUNK_COLS), jnp.float32),
            pltpu.VMEM((_CHUNK_ELEMS,), jnp.float32),
            pltpu.VMEM((_CHUNK_ELEMS,), jnp.float32),
            pltpu.VMEM((_SIDE_ELEMS,), jnp.float32),
            pltpu.SemaphoreType.DMA,
            pltpu.SemaphoreType.DMA,
            pltpu.SemaphoreType.DMA,
            pltpu.SemaphoreType.DMA,
        ],
        compiler_params=pltpu.CompilerParams(needs_layout_passes=False),
    )


_TOTAL_ROWS = _BATCH * _NUM_FIELDS
_ROWS_PER_W = _TOTAL_ROWS // _NW
_IDX_CHUNK = 128
_CHUNKS_PER_W = _ROWS_PER_W // _IDX_CHUNK
_FIRE = 8
_GROUPS = _CHUNKS_PER_W // _FIRE
_GROUP_ROWS = _FIRE * _IDX_CHUNK


def _gather_body(idx_hbm, table_hbm, out_hbm, idx_v, rows0, rows1,
                 sem, osem0, osem1):
    wid = lax.axis_index("s") * _NC + lax.axis_index("c")
    pltpu.sync_copy(idx_hbm.at[wid], idx_v)

    def out_at(g):
        return out_hbm.at[pl.ds(wid * _ROWS_PER_W + g * _GROUP_ROWS,
                                _GROUP_ROWS)]

    def wait_out(rows, osem):
        pltpu.make_async_copy(
            rows, out_hbm.at[pl.ds(0, _GROUP_ROWS)], osem).wait()

    def fire_and_drain(g, rows):
        copies = []
        for j in range(_FIRE):
            copies.append(
                pltpu.async_copy(
                    table_hbm.at[idx_v.at[g * _FIRE + j]],
                    rows.at[pl.ds(j * _IDX_CHUNK, _IDX_CHUNK)],
                    sem,
                )
            )
        for c in copies:
            c.wait()

    pltpu.async_copy(rows0, out_at(0), osem0)
    pltpu.async_copy(rows1, out_at(1), osem1)

    def pair(p, carry):
        g0 = 2 * p
        wait_out(rows0, osem0)
        fire_and_drain(g0, rows0)
        pltpu.async_copy(rows0, out_at(g0), osem0)
        wait_out(rows1, osem1)
        fire_and_drain(g0 + 1, rows1)
        pltpu.async_copy(rows1, out_at(g0 + 1), osem1)
        return carry

    lax.fori_loop(0, _GROUPS // 2, pair, 0)
    g_last = _GROUPS - 1
    wait_out(rows0, osem0)
    fire_and_drain(g_last, rows0)
    pltpu.async_copy(rows0, out_at(g_last), osem0)
    wait_out(rows0, osem0)
    wait_out(rows1, osem1)


@functools.cache
def _gather_call():
    return pl.kernel(
        _gather_body,
        mesh=plsc.VectorSubcoreMesh(core_axis_name="c", subcore_axis_name="s"),
        out_type=jax.ShapeDtypeStruct((_TOTAL_ROWS, _EMBED_DIM), jnp.float32),
        scratch_types=[
            pltpu.VMEM((_CHUNKS_PER_W, _IDX_CHUNK), jnp.int32),
            pltpu.VMEM((_GROUP_ROWS, _EMBED_DIM), jnp.float32),
            pltpu.VMEM((_GROUP_ROWS, _EMBED_DIM), jnp.float32),
            pltpu.SemaphoreType.DMA,
            pltpu.SemaphoreType.DMA,
            pltpu.SemaphoreType.DMA,
        ],
        compiler_params=pltpu.CompilerParams(use_tc_tiling_on_sc=False),
    )


def _mlp_body(dense_ref, emb_ref, w1a_ref, w1b_ref, b1_ref, w2_ref, b2_ref,
              w3_ref, b3_ref, wf_ref, bfw_ref, ww_ref, out_ref):
    f32 = jnp.float32
    hp = jax.lax.Precision.DEFAULT
    dense = dense_ref[...]
    h = jnp.dot(dense, w1a_ref[...], precision=hp, preferred_element_type=f32)
    h = h + jnp.dot(emb_ref[...], w1b_ref[...], precision=hp,
                    preferred_element_type=f32)
    h = jnp.maximum(h + b1_ref[...], 0.0)
    h = jnp.maximum(
        jnp.dot(h, w2_ref[...], precision=hp, preferred_element_type=f32)
        + b2_ref[...], 0.0)
    h = jnp.maximum(
        jnp.dot(h, w3_ref[...], precision=hp, preferred_element_type=f32)
        + b3_ref[...], 0.0)
    deep = jnp.sum(h * wf_ref[...], axis=1, keepdims=True)
    wide = jnp.sum(dense * ww_ref[...], axis=1, keepdims=True)
    z = 0.5 * (deep + wide + bfw_ref[...])
    out_ref[...] = jax.nn.sigmoid(z)


def _mlp_call(dense, emb_flat, w1a, w1b, b1, w2, b2, w3, b3, wfT, bfw, wwT):
    nblk = 8
    blk = _BATCH // nblk
    in_dim = _NUM_FIELDS * _EMBED_DIM
    full = lambda shape: pl.BlockSpec(shape, lambda i: (0, 0))
    return pl.pallas_call(
        _mlp_body,
        grid=(nblk,),
        in_specs=[
            pl.BlockSpec((blk, _NUM_DENSE), lambda i: (i, 0)),
            pl.BlockSpec((blk, in_dim), lambda i: (i, 0)),
            full((_NUM_DENSE, 256)),
            full((in_dim, 256)),
            full((1, 256)),
            full((256, 128)),
            full((1, 128)),
            full((128, 64)),
            full((1, 64)),
            full((1, 64)),
            full((1, 1)),
            full((1, _NUM_DENSE)),
        ],
        out_specs=pl.BlockSpec((blk, 1), lambda i: (i, 0)),
        out_shape=jax.ShapeDtypeStruct((_BATCH, 1), jnp.float32),
    )(dense, emb_flat, w1a, w1b, b1, w2, b2, w3, b3, wfT, bfw, wwT)


def kernel(dense_inputs, sparse_inputs, embed_tables,
           W1, b1, W2, b2, W3, b3, Wf, bf, Ww, bw):
    tblT = embed_tables.transpose(0, 2, 1).reshape(
        _NUM_FIELDS * _EMBED_DIM, _VOCAB)
    tail = embed_tables[:, _VOCAB_MAIN:, :].reshape(_SIDE_ELEMS)
    rm = _reformat_call()(tblT, tail)
    table = rm.reshape(_NUM_FIELDS * _VOCAB, _EMBED_DIM)

    offs = (jnp.arange(_NUM_FIELDS, dtype=jnp.int32) * _VOCAB)[None, :]
    idx = (sparse_inputs.astype(jnp.int32) + offs).reshape(
        _NW, _CHUNKS_PER_W, _IDX_CHUNK)

    emb_rows = _gather_call()(idx, table)
    emb_flat = emb_rows.reshape(_BATCH, _NUM_FIELDS * _EMBED_DIM)

    w1a = W1[:_NUM_DENSE]
    w1b = W1[_NUM_DENSE:]
    bfw = (bf + bw).reshape(1, 1)
    out = _mlp_call(dense_inputs, emb_flat,
                    w1a, w1b, b1.reshape(1, -1),
                    W2, b2.reshape(1, -1),
                    W3, b3.reshape(1, -1),
                    Wf.reshape(1, -1), bfw, Ww.reshape(1, -1))
    return out
